# baseline (speedup 1.0000x reference)
import functools

import jax
import jax.numpy as jnp
from jax import lax
from jax.experimental import pallas as pl
from jax.experimental.pallas import tpu as pltpu
from jax.experimental.pallas import tpu_sc as plsc

VOCAB = 100000
EMBED_DIM = 64
BATCH = 1024

_SC_INFO = plsc.get_sparse_core_info()
_NC = _SC_INFO.num_cores
_NS = _SC_INFO.num_subcores
_NW = _NC * _NS
_B_PER_W = BATCH // _NW

_V_BLK = 2048


def _make_gather():
  mesh = plsc.VectorSubcoreMesh(core_axis_name="c", subcore_axis_name="s")

  @functools.partial(
      pl.kernel,
      mesh=mesh,
      out_type=jax.ShapeDtypeStruct((BATCH, EMBED_DIM), jnp.float32),
      scratch_types=[
          pltpu.VMEM((_B_PER_W,), jnp.int32),
          pltpu.VMEM((_B_PER_W, EMBED_DIM), jnp.float32),
          pltpu.SemaphoreType.DMA,
      ],
      compiler_params=pltpu.CompilerParams(use_tc_tiling_on_sc=False),
  )
  def gather_kernel(table_hbm, idx_hbm, out_hbm, idx_v, rows_v, sem):
    wid = lax.axis_index("s") * _NC + lax.axis_index("c")
    base = wid * _B_PER_W
    pltpu.sync_copy(idx_hbm.at[pl.ds(base, _B_PER_W)], idx_v)
    pltpu.async_copy(table_hbm.at[idx_v], rows_v, sem).wait()
    pltpu.sync_copy(rows_v, out_hbm.at[pl.ds(base, _B_PER_W)])

  return gather_kernel


_gather = _make_gather()


def _matmul_body(e_ref, w_ref, b_ref, o_ref):
  o_ref[...] = (
      lax.dot_general(
          e_ref[...],
          w_ref[...],
          (((1,), (1,)), ((), ())),
          preferred_element_type=jnp.float32,
      )
      + b_ref[...]
  )


@jax.jit
def kernel(inputs, embeddings, W, b):
  embeds = _gather(embeddings, inputs.astype(jnp.int32))
  n_blk = pl.cdiv(VOCAB, _V_BLK)
  out = pl.pallas_call(
      _matmul_body,
      grid=(n_blk,),
      in_specs=[
          pl.BlockSpec((BATCH, EMBED_DIM), lambda i: (0, 0)),
          pl.BlockSpec((_V_BLK, EMBED_DIM), lambda i: (i, 0)),
          pl.BlockSpec((1, _V_BLK), lambda i: (0, i)),
      ],
      out_specs=pl.BlockSpec((BATCH, _V_BLK), lambda i: (0, i)),
      out_shape=jax.ShapeDtypeStruct((BATCH, VOCAB), jnp.float32),
  )(embeds, W, b.reshape(1, VOCAB))
  return out
